# Initial kernel scaffold; baseline (speedup 1.0000x reference)
import jax
import jax.numpy as jnp
from jax.experimental import pallas as pl


def kernel(positions, hidden_states, residual, Wq, Wk, Wv, Wo, q_norm_w, k_norm_w, post_attn_w, post_ff_w, gate_w, w_gate, w_up, w_down):
    raise NotImplementedError("write your pallas kernel here")


# speedup vs baseline: 1.2115x; 1.2115x over previous
import functools

import jax
import jax.numpy as jnp
from jax import lax
from jax.experimental import pallas as pl
from jax.experimental.pallas import tpu as pltpu
from jax.experimental.pallas import tpu_sc as plsc

_B, _S, _D, _H, _DH = 1, 2048, 768, 12, 64
_E, _TOPK, _FF = 8, 2, 512
_EPS = 1e-06
_THETA = 10000.0
_HALF = _DH // 2

_BT = 256
_NT = _S // _BT
_BQ = 512
_NQ = _S // _BQ


def _rms(x, w):
    v = jnp.mean(x * x, axis=-1, keepdims=True)
    return x * lax.rsqrt(v + _EPS) * w


def _qkv_body(x_ref, wq_ref, wk_ref, wv_ref, qn_ref, kn_ref, cos_ref, sin_ref,
              q_ref, k_ref, v_ref):
    x = x_ref[...]
    q = _rms(jnp.dot(x, wq_ref[...], preferred_element_type=jnp.float32), qn_ref[...])
    k = _rms(jnp.dot(x, wk_ref[...], preferred_element_type=jnp.float32), kn_ref[...])
    v_ref[...] = jnp.dot(x, wv_ref[...], preferred_element_type=jnp.float32)

    j = lax.broadcasted_iota(jnp.int32, (_BT, _D), 1) % _DH
    cos = cos_ref[...]
    ssin = sin_ref[...]

    def rope(t):
        partner = jnp.where(j < _HALF, jnp.roll(t, -_HALF, axis=1),
                            jnp.roll(t, _HALF, axis=1))
        return t * cos + partner * ssin

    q_ref[...] = rope(q)
    k_ref[...] = rope(k)


def _qkv(x, wq, wk, wv, qn, kn, cosf, sinf):
    out = [jax.ShapeDtypeStruct((_S, _D), jnp.float32)] * 3
    full = pl.BlockSpec((_D, _D), lambda i: (0, 0))
    wvec = pl.BlockSpec((1, _D), lambda i: (0, 0))
    blk = pl.BlockSpec((_BT, _D), lambda i: (i, 0))
    return pl.pallas_call(
        _qkv_body,
        grid=(_NT,),
        in_specs=[blk, full, full, full, wvec, wvec, blk, blk],
        out_specs=[blk, blk, blk],
        out_shape=out,
    )(x, wq, wk, wv, qn, kn, cosf, sinf)


def _rope_tables(positions):
    pos = positions.reshape(_S).astype(jnp.float32)
    inv = 1.0 / (_THETA ** (jnp.arange(_HALF, dtype=jnp.float32) / _HALF))
    ang = pos[:, None] * inv
    c = jnp.cos(ang)
    s = jnp.sin(ang)
    cosf = jnp.tile(jnp.concatenate([c, c], axis=1), (1, _H))
    sinf = jnp.tile(jnp.concatenate([-s, s], axis=1), (1, _H))
    return cosf, sinf


def _attn_body(q_ref, k_ref, v_ref, o_ref):
    qi = pl.program_id(1)
    row = qi * _BQ + lax.broadcasted_iota(jnp.int32, (_BQ, _S), 0)
    col = lax.broadcasted_iota(jnp.int32, (_BQ, _S), 1)
    q = q_ref[0]
    k = k_ref[0]
    s = lax.dot_general(q, k, (((1,), (1,)), ((), ())),
                        preferred_element_type=jnp.float32) * (1.0 / 8.0)
    s = jnp.where(col <= row, s, -1e30)
    m = jnp.max(s, axis=1, keepdims=True)
    p = jnp.exp(s - m)
    p = p / jnp.sum(p, axis=1, keepdims=True)
    o_ref[0] = jnp.dot(p, v_ref[0], preferred_element_type=jnp.float32)


def _attention(qh, kh, vh):
    qspec = pl.BlockSpec((1, _BQ, _DH), lambda h, qi: (h, qi, 0))
    kspec = pl.BlockSpec((1, _S, _DH), lambda h, qi: (h, 0, 0))
    return pl.pallas_call(
        _attn_body,
        grid=(_H, _NQ),
        in_specs=[qspec, kspec, kspec],
        out_specs=qspec,
        out_shape=jax.ShapeDtypeStruct((_H, _S, _DH), jnp.float32),
    )(qh, kh, vh)


_BK = 512
_NK = _S // _BK


def _attn2_body(q_ref, k_ref, v_ref, o_ref, acc_ref, m_ref, l_ref):
    qi = pl.program_id(1)
    ki = pl.program_id(2)

    @pl.when(ki == 0)
    def _():
        m_ref[...] = jnp.full_like(m_ref, -1e30)
        l_ref[...] = jnp.zeros_like(l_ref)
        acc_ref[...] = jnp.zeros_like(acc_ref)

    @pl.when(ki <= qi)
    def _():
        q = q_ref[0]
        k = k_ref[0]
        s = lax.dot_general(q, k, (((1,), (1,)), ((), ())),
                            preferred_element_type=jnp.float32) * (1.0 / 8.0)

        row = qi * _BQ + lax.broadcasted_iota(jnp.int32, (_BQ, _BK), 0)
        col = ki * _BK + lax.broadcasted_iota(jnp.int32, (_BQ, _BK), 1)
        s = jnp.where(col <= row, s, -1e30)
        m_old = m_ref[:, :1]
        m_new = jnp.maximum(m_old, jnp.max(s, axis=1, keepdims=True))
        p = jnp.exp(s - m_new)
        alpha = jnp.exp(m_old - m_new)
        l_ref[:, :1] = l_ref[:, :1] * alpha + jnp.sum(p, axis=1, keepdims=True)
        acc_ref[...] = acc_ref[...] * alpha + jnp.dot(
            p, v_ref[0], preferred_element_type=jnp.float32)
        m_ref[:, :1] = m_new

        @pl.when(ki == qi)
        def _():
            o_ref[0] = acc_ref[...] / l_ref[:, :1]


def _attention2(qh, kh, vh):
    qspec = pl.BlockSpec((1, _BQ, _DH), lambda h, qi, ki: (h, qi, 0))
    kspec = pl.BlockSpec((1, _BK, _DH), lambda h, qi, ki: (h, ki, 0))
    return pl.pallas_call(
        _attn2_body,
        grid=(_H, _NQ, _NK),
        in_specs=[qspec, kspec, kspec],
        out_specs=qspec,
        out_shape=jax.ShapeDtypeStruct((_H, _S, _DH), jnp.float32),
        scratch_shapes=[pltpu.VMEM((_BQ, _DH), jnp.float32),
                        pltpu.VMEM((_BQ, 128), jnp.float32),
                        pltpu.VMEM((_BQ, 128), jnp.float32)],
    )(qh, kh, vh)


def _oproj_body(ctx_ref, wo_ref, pw_ref, res_ref, h_ref):
    a = jnp.dot(ctx_ref[...], wo_ref[...], preferred_element_type=jnp.float32)
    h_ref[...] = _rms(a, pw_ref[...]) + res_ref[...]


def _oproj(ctx, wo, pw, res):
    blk = pl.BlockSpec((_BT, _D), lambda i: (i, 0))
    return pl.pallas_call(
        _oproj_body,
        grid=(_NT,),
        in_specs=[blk, pl.BlockSpec((_D, _D), lambda i: (0, 0)),
                  pl.BlockSpec((1, _D), lambda i: (0, 0)), blk],
        out_specs=blk,
        out_shape=jax.ShapeDtypeStruct((_S, _D), jnp.float32),
    )(ctx, wo, pw, res)


def _router_body(x_ref, gw_ref, w_ref, i1_ref, i2_ref, m1_ref, m2_ref):
    lane = lax.broadcasted_iota(jnp.int32, (_BT, 128), 1)
    logits = jnp.dot(x_ref[...], gw_ref[...], preferred_element_type=jnp.float32)
    logits = jnp.where(lane < _E, logits, -1e30)
    m = jnp.max(logits, axis=1, keepdims=True)
    p = jnp.exp(logits - m)
    p = p / jnp.sum(p, axis=1, keepdims=True)
    m1 = jnp.max(p, axis=1, keepdims=True)
    i1 = jnp.min(jnp.where(p == m1, lane, 9999), axis=1, keepdims=True)
    p2 = jnp.where(lane == i1, -1.0, p)
    m2 = jnp.max(p2, axis=1, keepdims=True)
    i2 = jnp.min(jnp.where(p2 == m2, lane, 9999), axis=1, keepdims=True)
    w_ref[...] = m1 * (lane == i1) + m2 * (lane == i2)
    i1_ref[...] = i1
    i2_ref[...] = i2
    m1_ref[...] = m1
    m2_ref[...] = m2


def _router(x, gw_pad):
    col_i = pl.BlockSpec((_BT, 1), lambda i: (i, 0))
    return pl.pallas_call(
        _router_body,
        grid=(_NT,),
        in_specs=[pl.BlockSpec((_BT, _D), lambda i: (i, 0)),
                  pl.BlockSpec((_D, 128), lambda i: (0, 0))],
        out_specs=[pl.BlockSpec((_BT, 128), lambda i: (i, 0)),
                   col_i, col_i, col_i, col_i],
        out_shape=[jax.ShapeDtypeStruct((_S, 128), jnp.float32),
                   jax.ShapeDtypeStruct((_S, 1), jnp.int32),
                   jax.ShapeDtypeStruct((_S, 1), jnp.int32),
                   jax.ShapeDtypeStruct((_S, 1), jnp.float32),
                   jax.ShapeDtypeStruct((_S, 1), jnp.float32)],
    )(x, gw_pad)


def _moe_body(x_ref, w_ref, wg_ref, wu_ref, wd_ref, o_ref):
    e = pl.program_id(1)
    x = x_ref[...]
    g = jnp.dot(x, wg_ref[0], preferred_element_type=jnp.float32)
    u = jnp.dot(x, wu_ref[0], preferred_element_type=jnp.float32)
    g = g * (1.0 / (1.0 + jnp.exp(-g))) * u
    y = jnp.dot(g, wd_ref[0], preferred_element_type=jnp.float32)
    lane = lax.broadcasted_iota(jnp.int32, (_S // 2, 128), 1)
    we = jnp.sum(jnp.where(lane == e, w_ref[...], 0.0), axis=1, keepdims=True)
    contrib = we * y

    @pl.when(e == 0)
    def _():
        o_ref[...] = contrib

    @pl.when(e > 0)
    def _():
        o_ref[...] += contrib


def _moe(x, wfull, w_gate, w_up, w_down):
    half = _S // 2
    tok = pl.BlockSpec((half, _D), lambda t, e: (t, 0))
    return pl.pallas_call(
        _moe_body,
        grid=(2, _E),
        in_specs=[tok,
                  pl.BlockSpec((half, 128), lambda t, e: (t, 0)),
                  pl.BlockSpec((1, _D, _FF), lambda t, e: (e, 0, 0)),
                  pl.BlockSpec((1, _D, _FF), lambda t, e: (e, 0, 0)),
                  pl.BlockSpec((1, _FF, _D), lambda t, e: (e, 0, 0))],
        out_specs=tok,
        out_shape=jax.ShapeDtypeStruct((_S, _D), jnp.float32),
    )(x, wfull, w_gate, w_up, w_down)


_NC, _NS, _L = 2, 16, 16
_NW = _NC * _NS
_NP = _S * _TOPK
_PPW = _NP // _NS
_GRW = _NP // _NW
_CTW = _S // _NW
_BM = 128
_NB = _NP // _BM
_NTILES = _NB + _E - 1


def _dispatch_body(eids_ref, pws_ref, tok_ref, wsr_ref, inv_ref, offs_ref,
                   ev_v, pw_v, hist_v, all_v, pos_v, tok_v, offs_v,
                   hist_sh, tok_sh, w_sh):
    c = lax.axis_index("c")
    w = lax.axis_index("s")
    lane = lax.iota(jnp.int32, _L)

    @pl.when(c == 0)
    def _():
        pltpu.sync_copy(eids_ref.at[pl.ds(w * _PPW, _PPW)], ev_v)
        pltpu.sync_copy(pws_ref.at[pl.ds(w * _PPW, _PPW)], pw_v)
        acc = jnp.zeros((_L,), jnp.int32)
        for r in range(_PPW // _L):
            evr = ev_v[pl.ds(r * _L, _L)]
            for e in range(_E):
                cnt = jnp.sum((evr == e).astype(jnp.int32))
                acc = acc + jnp.where(lane == e, cnt, 0)
        hist_v[...] = acc
        pltpu.sync_copy(hist_v, hist_sh.at[w])
        plsc.subcore_barrier()
        pltpu.sync_copy(hist_sh, all_v)
        pre = jnp.zeros((_L,), jnp.int32)
        tot = jnp.zeros((_L,), jnp.int32)
        for w2 in range(_NS):
            hrow = all_v[w2]
            tot = tot + hrow
            pre = pre + jnp.where(w2 < w, hrow, 0)
        offs = plsc.cumsum(tot) - tot
        base = offs + pre

        @pl.when(w == 0)
        def _():
            offs_v[...] = offs
            pltpu.sync_copy(offs_v, offs_ref)

        running = base
        for r in range(_PPW // _L):
            evr = ev_v[pl.ds(r * _L, _L)]
            gath = jnp.zeros((_L,), jnp.int32)
            rank = jnp.zeros((_L,), jnp.int32)
            cnts = jnp.zeros((_L,), jnp.int32)
            for e in range(_E):
                m = evr == e
                bsum = jnp.sum(jnp.where(lane == e, running, 0))
                gath = jnp.where(m, bsum, gath)
                cum = jnp.cumsum(m.astype(jnp.int32))
                rank = jnp.where(m, cum - 1, rank)
                cnt = jnp.sum(m.astype(jnp.int32))
                cnts = cnts + jnp.where(lane == e, cnt, 0)
            pos_v[r // 8, pl.ds((r % 8) * _L, _L)] = gath + rank
            running = running + cnts
        tokbase = w * _PPW - jnp.where(w >= _NS // 2, _S, 0)
        for r in range(_PPW // _L):
            tok_v[pl.ds(r * _L, _L)] = tokbase + r * _L + lane
        for j in range(2):
            pltpu.sync_copy(tok_v.at[pl.ds(j * 128, 128)], tok_sh.at[pos_v.at[j]])
            pltpu.sync_copy(pw_v.at[pl.ds(j * 128, 128)], w_sh.at[pos_v.at[j]])
            pltpu.sync_copy(pos_v.at[j],
                            inv_ref.at[pl.ds(w * _PPW + j * 128, 128)])
        plsc.subcore_barrier()
        sl = pl.ds(w * _PPW, _PPW)
        pltpu.sync_copy(tok_sh.at[sl], tok_ref.at[sl])
        pltpu.sync_copy(w_sh.at[sl], wsr_ref.at[sl])


def _sc_dispatch(eids, pws):
    mesh = plsc.VectorSubcoreMesh(core_axis_name="c", subcore_axis_name="s",
                                  num_cores=_NC, num_subcores=_NS)
    out_type = (jax.ShapeDtypeStruct((_NP,), jnp.int32),
                jax.ShapeDtypeStruct((_NP,), jnp.float32),
                jax.ShapeDtypeStruct((_NP,), jnp.int32),
                jax.ShapeDtypeStruct((_L,), jnp.int32))
    scratch = [pltpu.VMEM((_PPW,), jnp.int32),
               pltpu.VMEM((_PPW,), jnp.float32),
               pltpu.VMEM((_L,), jnp.int32),
               pltpu.VMEM((_NS, _L), jnp.int32),
               pltpu.VMEM((2, 128), jnp.int32),
               pltpu.VMEM((_PPW,), jnp.int32),
               pltpu.VMEM((_L,), jnp.int32),
               pltpu.VMEM_SHARED((_NS, _L), jnp.int32),
               pltpu.VMEM_SHARED((_NP,), jnp.int32),
               pltpu.VMEM_SHARED((_NP,), jnp.float32)]
    return pl.kernel(_dispatch_body, out_type=out_type, mesh=mesh,
                     scratch_types=scratch)(eids, pws)


def _gather_body(tok_ref, x_ref, xs_ref, idx_v, rows_v, sem):
    wid = lax.axis_index("s") * _NC + lax.axis_index("c")
    base = wid * _GRW
    pltpu.sync_copy(tok_ref.at[pl.ds(base, _GRW)], idx_v)
    pltpu.async_copy(x_ref.at[idx_v], rows_v, sem).wait()
    pltpu.sync_copy(rows_v, xs_ref.at[pl.ds(base, _GRW)])


def _sc_gather(tok, x):
    mesh = plsc.VectorSubcoreMesh(core_axis_name="c", subcore_axis_name="s",
                                  num_cores=_NC, num_subcores=_NS)
    scratch = [pltpu.VMEM((_GRW,), jnp.int32),
               pltpu.VMEM((_GRW, _D), jnp.float32),
               pltpu.SemaphoreType.DMA]
    return pl.kernel(_gather_body,
                     out_type=jax.ShapeDtypeStruct((_NP, _D), jnp.float32),
                     mesh=mesh, scratch_types=scratch)(tok, x)


def _combine_body(inv_ref, ys_ref, moe_ref, idx0_v, idx1_v, r0_v, r1_v,
                  sem0, sem1):
    wid = lax.axis_index("s") * _NC + lax.axis_index("c")
    t0 = wid * _CTW
    pltpu.sync_copy(inv_ref.at[pl.ds(t0, _CTW)], idx0_v)
    pltpu.sync_copy(inv_ref.at[pl.ds(_S + t0, _CTW)], idx1_v)
    cp0 = pltpu.async_copy(ys_ref.at[idx0_v], r0_v, sem0)
    cp1 = pltpu.async_copy(ys_ref.at[idx1_v], r1_v, sem1)
    cp0.wait()
    cp1.wait()

    def body(i, carry):
        for ch in range(_D // _L):
            sl = pl.ds(ch * _L, _L)
            r0_v[i, sl] = r0_v[i, sl] + r1_v[i, sl]
        return carry

    lax.fori_loop(0, _CTW, body, 0)
    pltpu.sync_copy(r0_v, moe_ref.at[pl.ds(t0, _CTW)])


def _sc_combine(inv, ys):
    mesh = plsc.VectorSubcoreMesh(core_axis_name="c", subcore_axis_name="s",
                                  num_cores=_NC, num_subcores=_NS)
    scratch = [pltpu.VMEM((_CTW,), jnp.int32),
               pltpu.VMEM((_CTW,), jnp.int32),
               pltpu.VMEM((_CTW, _D), jnp.float32),
               pltpu.VMEM((_CTW, _D), jnp.float32),
               pltpu.SemaphoreType.DMA,
               pltpu.SemaphoreType.DMA]
    return pl.kernel(_combine_body,
                     out_type=jax.ShapeDtypeStruct((_S, _D), jnp.float32),
                     mesh=mesh, scratch_types=scratch)(inv, ys)


def _gmm_body(meta_ref, x_ref, w_ref, wg_ref, wu_ref, wd_ref, o_ref):
    i = pl.program_id(0)

    @pl.when(meta_ref[5, i] == 1)
    def _():
        rs = meta_ref[2, i]
        re_ = meta_ref[3, i]
        g0 = meta_ref[0, i] * _BM
        rows = g0 + lax.broadcasted_iota(jnp.int32, (_BM, _D), 0)
        x = jnp.where((rows >= rs) & (rows < re_), x_ref[...], 0.0)
        g = jnp.dot(x, wg_ref[0], preferred_element_type=jnp.float32)
        u = jnp.dot(x, wu_ref[0], preferred_element_type=jnp.float32)
        gg = g * (1.0 / (1.0 + jnp.exp(-g))) * u
        y = jnp.dot(gg, wd_ref[0], preferred_element_type=jnp.float32) * w_ref[...]

        @pl.when(meta_ref[4, i] == 1)
        def _():
            o_ref[...] = y

        @pl.when(meta_ref[4, i] == 0)
        def _():
            o_ref[...] += y


def _gmm(meta, xs, wsr, w_gate, w_up, w_down):
    grid_spec = pltpu.PrefetchScalarGridSpec(
        num_scalar_prefetch=1,
        grid=(_NTILES,),
        in_specs=[
            pl.BlockSpec((_BM, _D), lambda i, m: (m[0, i], 0)),
            pl.BlockSpec((_BM, 1), lambda i, m: (m[0, i], 0)),
            pl.BlockSpec((1, _D, _FF), lambda i, m: (m[1, i], 0, 0)),
            pl.BlockSpec((1, _D, _FF), lambda i, m: (m[1, i], 0, 0)),
            pl.BlockSpec((1, _FF, _D), lambda i, m: (m[1, i], 0, 0)),
        ],
        out_specs=pl.BlockSpec((_BM, _D), lambda i, m: (m[0, i], 0)),
    )
    return pl.pallas_call(
        _gmm_body, grid_spec=grid_spec,
        out_shape=jax.ShapeDtypeStruct((_NP, _D), jnp.float32),
    )(meta, xs, wsr.reshape(_NP, 1), w_gate, w_up, w_down)


def _tile_meta(offs16):
    starts = offs16[:_E]
    ends = offs16[1:_E + 1]
    blo = (jnp.arange(_NB, dtype=jnp.int32) * _BM)[:, None]
    bhi = blo + _BM
    rs = jnp.maximum(starts[None, :], blo)
    re = jnp.minimum(ends[None, :], bhi)
    vflat = (rs < re).reshape(-1)
    order = jnp.argsort(~vflat, stable=True)
    sel = order[:_NTILES]
    bids = jnp.repeat(jnp.arange(_NB, dtype=jnp.int32), _E)[sel]
    eids = jnp.tile(jnp.arange(_E, dtype=jnp.int32), _NB)[sel]
    rss = rs.reshape(-1)[sel]
    ress = re.reshape(-1)[sel]
    vs = vflat[sel]
    nlast = jnp.clip(jnp.sum(vflat.astype(jnp.int32)) - 1, 0, _NTILES - 1)
    bids = jnp.where(vs, bids, bids[nlast])
    eids = jnp.where(vs, eids, eids[nlast])
    rss = jnp.where(vs, rss, 0)
    ress = jnp.where(vs, ress, 0)
    prevb = jnp.concatenate([jnp.array([-1], dtype=jnp.int32), bids[:-1]])
    init = (vs & (bids != prevb)).astype(jnp.int32)
    return jnp.stack([bids, eids, rss, ress, init,
                      vs.astype(jnp.int32)]).astype(jnp.int32)


def _moe_sc(h, i1, i2, m1, m2, w_gate, w_up, w_down):
    eids = jnp.concatenate([i1.reshape(_S), i2.reshape(_S)])
    pws = jnp.concatenate([m1.reshape(_S), m2.reshape(_S)])
    tok, wsr, inv, offs16 = _sc_dispatch(eids, pws)
    xs = _sc_gather(tok, h)
    meta = _tile_meta(offs16)
    ys = _gmm(meta, xs, wsr, w_gate, w_up, w_down)
    return _sc_combine(inv, ys)


def _final_body(moe_ref, pw_ref, h_ref, o_ref):
    o_ref[...] = _rms(moe_ref[...], pw_ref[...]) + h_ref[...]


def _final(moe, pw, h):
    blk = pl.BlockSpec((_BT, _D), lambda i: (i, 0))
    return pl.pallas_call(
        _final_body,
        grid=(_NT,),
        in_specs=[blk, pl.BlockSpec((1, _D), lambda i: (0, 0)), blk],
        out_specs=blk,
        out_shape=jax.ShapeDtypeStruct((_S, _D), jnp.float32),
    )(moe, pw, h)


_USE = dict(qkv=True, attn=True, oproj=True, router=True, moe=True, final=True)


def _xla_rms(x, w):
    v = jnp.mean(jnp.square(x), axis=-1, keepdims=True)
    return (x * lax.rsqrt(v + _EPS)) * w


def _xla_rope(x):
    half = _HALF
    pos = jnp.arange(_S, dtype=jnp.float32)
    inv = 1.0 / (_THETA ** (jnp.arange(half, dtype=jnp.float32) / half))
    ang = pos[:, None] * inv
    cos = jnp.cos(ang)[:, None, :]
    sin = jnp.sin(ang)[:, None, :]
    x = x.reshape(_S, _H, _DH)
    x1, x2 = x[..., :half], x[..., half:]
    r = jnp.concatenate([x1 * cos - x2 * sin, x2 * cos + x1 * sin], axis=-1)
    return r.reshape(_S, _D)


def kernel(positions, hidden_states, residual, Wq, Wk, Wv, Wo, q_norm_w,
           k_norm_w, post_attn_w, post_ff_w, gate_w, w_gate, w_up, w_down):
    del residual
    x = hidden_states.reshape(_S, _D)
    cosf, sinf = _rope_tables(positions)
    qn = q_norm_w.reshape(1, _D)
    kn = k_norm_w.reshape(1, _D)
    paw = post_attn_w.reshape(1, _D)
    pfw = post_ff_w.reshape(1, _D)
    gw_pad = jnp.pad(gate_w, ((0, 0), (0, 128 - _E)))

    if _USE["qkv"]:
        q, k, v = _qkv(x, Wq, Wk, Wv, qn, kn, cosf, sinf)
    else:
        q = _xla_rope(_xla_rms(x @ Wq, q_norm_w))
        k = _xla_rope(_xla_rms(x @ Wk, k_norm_w))
        v = x @ Wv

    if _USE["attn"]:
        qh = q.reshape(_S, _H, _DH).transpose(1, 0, 2)
        kh = k.reshape(_S, _H, _DH).transpose(1, 0, 2)
        vh = v.reshape(_S, _H, _DH).transpose(1, 0, 2)
        ctx = _attention(qh, kh, vh).transpose(1, 0, 2).reshape(_S, _D)
    else:
        q4 = q.reshape(_S, _H, _DH)
        k4 = k.reshape(_S, _H, _DH)
        v4 = v.reshape(_S, _H, _DH)
        scores = jnp.einsum('qhd,khd->hqk', q4, k4) / 8.0
        mask = jnp.tril(jnp.ones((_S, _S), dtype=bool))
        scores = jnp.where(mask[None, :, :], scores, jnp.finfo(jnp.float32).min)
        attnp = jax.nn.softmax(scores, axis=-1)
        ctx = jnp.einsum('hqk,khd->qhd', attnp, v4).reshape(_S, _D)

    if _USE["oproj"]:
        h = _oproj(ctx, Wo, paw, x)
    else:
        h = _xla_rms(ctx @ Wo, post_attn_w) + x

    if _USE["router"]:
        wpad128, i1c, i2c, m1c, m2c = _router(h, gw_pad)
        wfull = wpad128[:, :_E]
    else:
        probs = jax.nn.softmax((h @ gate_w).astype(jnp.float32), axis=-1)
        topw, topi = jax.lax.top_k(probs, _TOPK)
        wfull = jnp.zeros((_S, _E), dtype=jnp.float32)
        for kk in range(_TOPK):
            wfull = wfull + jax.nn.one_hot(topi[:, kk], _E) * topw[:, kk:kk + 1]
    wpad = jnp.pad(wfull, ((0, 0), (0, 128 - _E)))

    if _USE["moe"] == "sc":
        moe = _moe_sc(h, i1c, i2c, m1c, m2c, w_gate, w_up, w_down)
    elif _USE["moe"]:
        moe = _moe(h, wpad, w_gate, w_up, w_down)
    else:
        out = jnp.zeros_like(h)
        for e in range(_E):
            g = jax.nn.silu(h @ w_gate[e]) * (h @ w_up[e])
            out = out + wfull[:, e:e + 1] * (g @ w_down[e])
        moe = out

    if _USE["final"]:
        out = _final(moe, pfw, h)
    else:
        out = _xla_rms(moe, post_ff_w) + h
    return out.reshape(_B, _S, _D)


# speedup vs baseline: 1.2189x; 1.0061x over previous
import functools

import jax
import jax.numpy as jnp
from jax import lax
from jax.experimental import pallas as pl
from jax.experimental.pallas import tpu as pltpu
from jax.experimental.pallas import tpu_sc as plsc

_B, _S, _D, _H, _DH = 1, 2048, 768, 12, 64
_E, _TOPK, _FF = 8, 2, 512
_EPS = 1e-06
_THETA = 10000.0
_HALF = _DH // 2

_BT = 256
_NT = _S // _BT
_BQ = 512
_NQ = _S // _BQ


def _rms(x, w):
    v = jnp.mean(x * x, axis=-1, keepdims=True)
    return x * lax.rsqrt(v + _EPS) * w


def _qkv_body(x_ref, wq_ref, wk_ref, wv_ref, qn_ref, kn_ref, cos_ref, sin_ref,
              q_ref, k_ref, v_ref):
    x = x_ref[...]
    q = _rms(jnp.dot(x, wq_ref[...], preferred_element_type=jnp.float32), qn_ref[...])
    k = _rms(jnp.dot(x, wk_ref[...], preferred_element_type=jnp.float32), kn_ref[...])
    v_ref[...] = jnp.dot(x, wv_ref[...], preferred_element_type=jnp.float32)

    j = lax.broadcasted_iota(jnp.int32, (_BT, _D), 1) % _DH
    cos = cos_ref[...]
    ssin = sin_ref[...]

    def rope(t):
        partner = jnp.where(j < _HALF, jnp.roll(t, -_HALF, axis=1),
                            jnp.roll(t, _HALF, axis=1))
        return t * cos + partner * ssin

    q_ref[...] = rope(q)
    k_ref[...] = rope(k)


def _qkv(x, wq, wk, wv, qn, kn, cosf, sinf):
    out = [jax.ShapeDtypeStruct((_S, _D), jnp.float32)] * 3
    full = pl.BlockSpec((_D, _D), lambda i: (0, 0))
    wvec = pl.BlockSpec((1, _D), lambda i: (0, 0))
    blk = pl.BlockSpec((_BT, _D), lambda i: (i, 0))
    return pl.pallas_call(
        _qkv_body,
        grid=(_NT,),
        in_specs=[blk, full, full, full, wvec, wvec, blk, blk],
        out_specs=[blk, blk, blk],
        out_shape=out,
    )(x, wq, wk, wv, qn, kn, cosf, sinf)


def _rope_tables(positions):
    pos = positions.reshape(_S).astype(jnp.float32)
    inv = 1.0 / (_THETA ** (jnp.arange(_HALF, dtype=jnp.float32) / _HALF))
    ang = pos[:, None] * inv
    c = jnp.cos(ang)
    s = jnp.sin(ang)
    cosf = jnp.tile(jnp.concatenate([c, c], axis=1), (1, _H))
    sinf = jnp.tile(jnp.concatenate([-s, s], axis=1), (1, _H))
    return cosf, sinf


def _attn_body(q_ref, k_ref, v_ref, o_ref):
    qi = pl.program_id(1)
    row = qi * _BQ + lax.broadcasted_iota(jnp.int32, (_BQ, _S), 0)
    col = lax.broadcasted_iota(jnp.int32, (_BQ, _S), 1)
    q = q_ref[0]
    k = k_ref[0]
    s = lax.dot_general(q, k, (((1,), (1,)), ((), ())),
                        preferred_element_type=jnp.float32) * (1.0 / 8.0)
    s = jnp.where(col <= row, s, -1e30)
    m = jnp.max(s, axis=1, keepdims=True)
    p = jnp.exp(s - m)
    p = p / jnp.sum(p, axis=1, keepdims=True)
    o_ref[0] = jnp.dot(p, v_ref[0], preferred_element_type=jnp.float32)


def _attention(qh, kh, vh):
    qspec = pl.BlockSpec((1, _BQ, _DH), lambda h, qi: (h, qi, 0))
    kspec = pl.BlockSpec((1, _S, _DH), lambda h, qi: (h, 0, 0))
    return pl.pallas_call(
        _attn_body,
        grid=(_H, _NQ),
        in_specs=[qspec, kspec, kspec],
        out_specs=qspec,
        out_shape=jax.ShapeDtypeStruct((_H, _S, _DH), jnp.float32),
    )(qh, kh, vh)


_BK = 512
_NK = _S // _BK


def _attn2_body(q_ref, k_ref, v_ref, o_ref, acc_ref, m_ref, l_ref):
    qi = pl.program_id(1)
    ki = pl.program_id(2)

    @pl.when(ki == 0)
    def _():
        m_ref[...] = jnp.full_like(m_ref, -1e30)
        l_ref[...] = jnp.zeros_like(l_ref)
        acc_ref[...] = jnp.zeros_like(acc_ref)

    @pl.when(ki <= qi)
    def _():
        q = q_ref[0]
        k = k_ref[0]
        s = lax.dot_general(q, k, (((1,), (1,)), ((), ())),
                            preferred_element_type=jnp.float32) * (1.0 / 8.0)

        row = qi * _BQ + lax.broadcasted_iota(jnp.int32, (_BQ, _BK), 0)
        col = ki * _BK + lax.broadcasted_iota(jnp.int32, (_BQ, _BK), 1)
        s = jnp.where(col <= row, s, -1e30)
        m_old = m_ref[:, :1]
        m_new = jnp.maximum(m_old, jnp.max(s, axis=1, keepdims=True))
        p = jnp.exp(s - m_new)
        alpha = jnp.exp(m_old - m_new)
        l_ref[:, :1] = l_ref[:, :1] * alpha + jnp.sum(p, axis=1, keepdims=True)
        acc_ref[...] = acc_ref[...] * alpha + jnp.dot(
            p, v_ref[0], preferred_element_type=jnp.float32)
        m_ref[:, :1] = m_new

        @pl.when(ki == qi)
        def _():
            o_ref[0] = acc_ref[...] / l_ref[:, :1]


def _attention2(qh, kh, vh):
    qspec = pl.BlockSpec((1, _BQ, _DH), lambda h, qi, ki: (h, qi, 0))
    kspec = pl.BlockSpec((1, _BK, _DH), lambda h, qi, ki: (h, ki, 0))
    return pl.pallas_call(
        _attn2_body,
        grid=(_H, _NQ, _NK),
        in_specs=[qspec, kspec, kspec],
        out_specs=qspec,
        out_shape=jax.ShapeDtypeStruct((_H, _S, _DH), jnp.float32),
        scratch_shapes=[pltpu.VMEM((_BQ, _DH), jnp.float32),
                        pltpu.VMEM((_BQ, 128), jnp.float32),
                        pltpu.VMEM((_BQ, 128), jnp.float32)],
    )(qh, kh, vh)


def _oproj_body(ctx_ref, wo_ref, pw_ref, res_ref, h_ref):
    a = jnp.dot(ctx_ref[...], wo_ref[...], preferred_element_type=jnp.float32)
    h_ref[...] = _rms(a, pw_ref[...]) + res_ref[...]


def _oproj(ctx, wo, pw, res):
    blk = pl.BlockSpec((_BT, _D), lambda i: (i, 0))
    return pl.pallas_call(
        _oproj_body,
        grid=(_NT,),
        in_specs=[blk, pl.BlockSpec((_D, _D), lambda i: (0, 0)),
                  pl.BlockSpec((1, _D), lambda i: (0, 0)), blk],
        out_specs=blk,
        out_shape=jax.ShapeDtypeStruct((_S, _D), jnp.float32),
    )(ctx, wo, pw, res)


def _router_body(x_ref, gw_ref, w_ref, i1_ref, i2_ref, m1_ref, m2_ref):
    lane = lax.broadcasted_iota(jnp.int32, (_BT, 128), 1)
    logits = jnp.dot(x_ref[...], gw_ref[...], preferred_element_type=jnp.float32)
    logits = jnp.where(lane < _E, logits, -1e30)
    m = jnp.max(logits, axis=1, keepdims=True)
    p = jnp.exp(logits - m)
    p = p / jnp.sum(p, axis=1, keepdims=True)
    m1 = jnp.max(p, axis=1, keepdims=True)
    i1 = jnp.min(jnp.where(p == m1, lane, 9999), axis=1, keepdims=True)
    p2 = jnp.where(lane == i1, -1.0, p)
    m2 = jnp.max(p2, axis=1, keepdims=True)
    i2 = jnp.min(jnp.where(p2 == m2, lane, 9999), axis=1, keepdims=True)
    w_ref[...] = m1 * (lane == i1) + m2 * (lane == i2)
    i1_ref[...] = i1
    i2_ref[...] = i2
    m1_ref[...] = m1
    m2_ref[...] = m2


def _router(x, gw_pad):
    col_i = pl.BlockSpec((_BT, 1), lambda i: (i, 0))
    return pl.pallas_call(
        _router_body,
        grid=(_NT,),
        in_specs=[pl.BlockSpec((_BT, _D), lambda i: (i, 0)),
                  pl.BlockSpec((_D, 128), lambda i: (0, 0))],
        out_specs=[pl.BlockSpec((_BT, 128), lambda i: (i, 0)),
                   col_i, col_i, col_i, col_i],
        out_shape=[jax.ShapeDtypeStruct((_S, 128), jnp.float32),
                   jax.ShapeDtypeStruct((_S, 1), jnp.int32),
                   jax.ShapeDtypeStruct((_S, 1), jnp.int32),
                   jax.ShapeDtypeStruct((_S, 1), jnp.float32),
                   jax.ShapeDtypeStruct((_S, 1), jnp.float32)],
    )(x, gw_pad)


def _moe_body(x_ref, w_ref, wg_ref, wu_ref, wd_ref, o_ref):
    e = pl.program_id(1)
    x = x_ref[...]
    g = jnp.dot(x, wg_ref[0], preferred_element_type=jnp.float32)
    u = jnp.dot(x, wu_ref[0], preferred_element_type=jnp.float32)
    g = g * (1.0 / (1.0 + jnp.exp(-g))) * u
    y = jnp.dot(g, wd_ref[0], preferred_element_type=jnp.float32)
    lane = lax.broadcasted_iota(jnp.int32, (_S // 2, 128), 1)
    we = jnp.sum(jnp.where(lane == e, w_ref[...], 0.0), axis=1, keepdims=True)
    contrib = we * y

    @pl.when(e == 0)
    def _():
        o_ref[...] = contrib

    @pl.when(e > 0)
    def _():
        o_ref[...] += contrib


def _moe(x, wfull, w_gate, w_up, w_down):
    half = _S // 2
    tok = pl.BlockSpec((half, _D), lambda t, e: (t, 0))
    return pl.pallas_call(
        _moe_body,
        grid=(2, _E),
        in_specs=[tok,
                  pl.BlockSpec((half, 128), lambda t, e: (t, 0)),
                  pl.BlockSpec((1, _D, _FF), lambda t, e: (e, 0, 0)),
                  pl.BlockSpec((1, _D, _FF), lambda t, e: (e, 0, 0)),
                  pl.BlockSpec((1, _FF, _D), lambda t, e: (e, 0, 0))],
        out_specs=tok,
        out_shape=jax.ShapeDtypeStruct((_S, _D), jnp.float32),
    )(x, wfull, w_gate, w_up, w_down)


_NC, _NS, _L = 2, 16, 16
_NW = _NC * _NS
_NP = _S * _TOPK
_PPW = _NP // _NS
_GRW = _NP // _NW
_CTW = _S // _NW
_BM = 128
_NB = _NP // _BM
_NTILES = _NB + _E - 1


def _pos_body(eids_ref, inv_ref, offs_ref, ev_v, all_v, pos0_v, pos1_v,
              offs_v, buf_v, hist_sh):
    c = lax.axis_index("c")
    w = lax.axis_index("s")
    lane = lax.iota(jnp.int32, _L)

    def cumsum16(x):
        buf_v[pl.ds(_L, _L)] = x
        for k in (1, 2, 4, 8):
            x = x + buf_v[pl.ds(_L - k, _L)]
            buf_v[pl.ds(_L, _L)] = x
        return x

    @pl.when(c == 0)
    def _():
        buf_v[pl.ds(0, _L)] = jnp.zeros((_L,), jnp.int32)
        pltpu.sync_copy(eids_ref.at[pl.ds(w * _PPW, _PPW)], ev_v)
        lane_onehot = [1 - jnp.minimum(jnp.abs(lane - e), 1) for e in range(_E)]
        acc = jnp.zeros((_L,), jnp.int32)
        for r in range(_PPW // _L):
            evr = ev_v[pl.ds(r * _L, _L)]
            for e in range(_E):
                me = 1 - jnp.minimum(jnp.abs(evr - e), 1)
                cum = cumsum16(me)
                acc = acc + lane_onehot[e] * cum[_L - 1]
        buf_v[pl.ds(0, _L)] = acc
        pltpu.sync_copy(buf_v.at[pl.ds(0, _L)], hist_sh.at[w])
        buf_v[pl.ds(0, _L)] = jnp.zeros((_L,), jnp.int32)
        plsc.subcore_barrier()
        pltpu.sync_copy(hist_sh, all_v)
        pre = jnp.zeros((_L,), jnp.int32)
        tot = jnp.zeros((_L,), jnp.int32)
        for w2 in range(_NS):
            hrow = all_v[w2]
            tot = tot + hrow
            pre = pre + jnp.where(w2 < w, hrow, 0)
        offs = cumsum16(tot) - tot
        base = offs + pre

        @pl.when(w == 0)
        def _():
            offs_v[...] = offs
            pltpu.sync_copy(offs_v, offs_ref)

        running = base
        for r in range(_PPW // _L):
            evr = ev_v[pl.ds(r * _L, _L)]
            pos = jnp.zeros((_L,), jnp.int32)
            for e in range(_E):
                me = 1 - jnp.minimum(jnp.abs(evr - e), 1)
                cum = cumsum16(me)
                pos = pos + me * (running[e] + cum - 1)
                running = running + lane_onehot[e] * cum[_L - 1]
            if r < 8:
                pos0_v[pl.ds(r * _L, _L)] = pos
            else:
                pos1_v[pl.ds((r - 8) * _L, _L)] = pos
        for j, pos_ref in ((0, pos0_v), (1, pos1_v)):
            pltpu.sync_copy(pos_ref,
                            inv_ref.at[pl.ds(w * _PPW + j * 128, 128)])


def _sc_pos(eids):
    mesh = plsc.VectorSubcoreMesh(core_axis_name="c", subcore_axis_name="s",
                                  num_cores=_NC, num_subcores=_NS)
    out_type = (jax.ShapeDtypeStruct((_NP,), jnp.int32),
                jax.ShapeDtypeStruct((_L,), jnp.int32))
    scratch = [pltpu.VMEM((_PPW,), jnp.int32),
               pltpu.VMEM((_NS, _L), jnp.int32),
               pltpu.VMEM((128,), jnp.int32),
               pltpu.VMEM((128,), jnp.int32),
               pltpu.VMEM((_L,), jnp.int32),
               pltpu.VMEM((2 * _L,), jnp.int32),
               pltpu.VMEM_SHARED((_NS, _L), jnp.int32)]
    return pl.kernel(_pos_body, out_type=out_type, mesh=mesh,
                     scratch_types=scratch)(eids)


def _scatter_body(inv_ref, x_ref, xs_ref, pos_v, rows_v, sem):
    c = lax.axis_index("c")
    w = lax.axis_index("s")

    @pl.when(c == 0)
    def _():
        tokbase = w * _PPW - jnp.where(w >= _NS // 2, _S, 0)
        for j in range(2):
            pltpu.sync_copy(inv_ref.at[pl.ds(w * _PPW + j * 128, 128)], pos_v)
            pltpu.sync_copy(x_ref.at[pl.ds(tokbase + j * 128, 128)], rows_v)

            def scat_step(rc, carry):
                posvec = pos_v[pl.ds(rc * _L, _L)]
                cps = []
                for l in range(_L):
                    cps.append(pltpu.async_copy(
                        rows_v.at[pl.ds(rc * _L + l, 1)],
                        xs_ref.at[pl.ds(posvec[l], 1)], sem))
                for cp in cps:
                    cp.wait()
                return carry

            lax.fori_loop(0, 128 // _L, scat_step, 0)


def _sc_scatter(inv, x):
    mesh = plsc.VectorSubcoreMesh(core_axis_name="c", subcore_axis_name="s",
                                  num_cores=_NC, num_subcores=_NS)
    scratch = [pltpu.VMEM((128,), jnp.int32),
               pltpu.VMEM((128, _D), jnp.float32),
               pltpu.SemaphoreType.DMA]
    return pl.kernel(_scatter_body,
                     out_type=jax.ShapeDtypeStruct((_NP, _D), jnp.float32),
                     mesh=mesh, scratch_types=scratch)(inv, x)


def _combine_body(inv_ref, ys_ref, y0_ref, y1_ref, idx0_v, idx1_v, r0_v, r1_v,
                  sem0, sem1):
    wid = lax.axis_index("s") * _NC + lax.axis_index("c")
    t0 = wid * _CTW
    pltpu.sync_copy(inv_ref.at[pl.ds(t0, _CTW)], idx0_v)
    pltpu.sync_copy(inv_ref.at[pl.ds(_S + t0, _CTW)], idx1_v)
    cp0 = pltpu.async_copy(ys_ref.at[idx0_v], r0_v, sem0)
    cp1 = pltpu.async_copy(ys_ref.at[idx1_v], r1_v, sem1)
    cp0.wait()
    cp1.wait()
    pltpu.sync_copy(r0_v, y0_ref.at[pl.ds(t0, _CTW)])
    pltpu.sync_copy(r1_v, y1_ref.at[pl.ds(t0, _CTW)])


def _sc_combine(inv, ys):
    mesh = plsc.VectorSubcoreMesh(core_axis_name="c", subcore_axis_name="s",
                                  num_cores=_NC, num_subcores=_NS)
    scratch = [pltpu.VMEM((_CTW,), jnp.int32),
               pltpu.VMEM((_CTW,), jnp.int32),
               pltpu.VMEM((_CTW, _D), jnp.float32),
               pltpu.VMEM((_CTW, _D), jnp.float32),
               pltpu.SemaphoreType.DMA,
               pltpu.SemaphoreType.DMA]
    return pl.kernel(_combine_body,
                     out_type=(jax.ShapeDtypeStruct((_S, _D), jnp.float32),
                               jax.ShapeDtypeStruct((_S, _D), jnp.float32)),
                     mesh=mesh, scratch_types=scratch)(inv, ys)


def _gmm_body(meta_ref, x_ref, wg_ref, wu_ref, wd_ref, o_ref):
    i = pl.program_id(0)

    @pl.when(meta_ref[5, i] == 1)
    def _():
        rs = meta_ref[2, i]
        re_ = meta_ref[3, i]
        g0 = meta_ref[0, i] * _BM
        rows = g0 + lax.broadcasted_iota(jnp.int32, (_BM, _D), 0)
        x = jnp.where((rows >= rs) & (rows < re_), x_ref[...], 0.0)
        g = jnp.dot(x, wg_ref[0], preferred_element_type=jnp.float32)
        u = jnp.dot(x, wu_ref[0], preferred_element_type=jnp.float32)
        gg = g * (1.0 / (1.0 + jnp.exp(-g))) * u
        y = jnp.dot(gg, wd_ref[0], preferred_element_type=jnp.float32)

        @pl.when(meta_ref[4, i] == 1)
        def _():
            o_ref[...] = y

        @pl.when(meta_ref[4, i] == 0)
        def _():
            o_ref[...] += y


def _gmm(meta, xs, w_gate, w_up, w_down):
    grid_spec = pltpu.PrefetchScalarGridSpec(
        num_scalar_prefetch=1,
        grid=(_NTILES,),
        in_specs=[
            pl.BlockSpec((_BM, _D), lambda i, m: (m[0, i], 0)),
            pl.BlockSpec((1, _D, _FF), lambda i, m: (m[1, i], 0, 0)),
            pl.BlockSpec((1, _D, _FF), lambda i, m: (m[1, i], 0, 0)),
            pl.BlockSpec((1, _FF, _D), lambda i, m: (m[1, i], 0, 0)),
        ],
        out_specs=pl.BlockSpec((_BM, _D), lambda i, m: (m[0, i], 0)),
    )
    return pl.pallas_call(
        _gmm_body, grid_spec=grid_spec,
        out_shape=jax.ShapeDtypeStruct((_NP, _D), jnp.float32),
    )(meta, xs, w_gate, w_up, w_down)


def _tile_meta(offs16):
    starts = offs16[:_E]
    ends = offs16[1:_E + 1]
    blo = (jnp.arange(_NB, dtype=jnp.int32) * _BM)[:, None]
    bhi = blo + _BM
    rs = jnp.maximum(starts[None, :], blo)
    re = jnp.minimum(ends[None, :], bhi)
    vflat = (rs < re).reshape(-1)
    order = jnp.argsort(~vflat, stable=True)
    sel = order[:_NTILES]
    bids = jnp.repeat(jnp.arange(_NB, dtype=jnp.int32), _E)[sel]
    eids = jnp.tile(jnp.arange(_E, dtype=jnp.int32), _NB)[sel]
    rss = rs.reshape(-1)[sel]
    ress = re.reshape(-1)[sel]
    vs = vflat[sel]
    nlast = jnp.clip(jnp.sum(vflat.astype(jnp.int32)) - 1, 0, _NTILES - 1)
    bids = jnp.where(vs, bids, bids[nlast])
    eids = jnp.where(vs, eids, eids[nlast])
    rss = jnp.where(vs, rss, 0)
    ress = jnp.where(vs, ress, 0)
    prevb = jnp.concatenate([jnp.array([-1], dtype=jnp.int32), bids[:-1]])
    init = (vs & (bids != prevb)).astype(jnp.int32)
    return jnp.stack([bids, eids, rss, ress, init,
                      vs.astype(jnp.int32)]).astype(jnp.int32)


def _moe_sc(h, i1, i2, w_gate, w_up, w_down):
    eids = jnp.concatenate([i1.reshape(_S), i2.reshape(_S)])
    inv, offs16 = _sc_pos(eids)
    xs = _sc_scatter(inv, h)
    meta = _tile_meta(offs16)
    ys = _gmm(meta, xs, w_gate, w_up, w_down)
    return _sc_combine(inv, ys)


def _final_body(moe_ref, pw_ref, h_ref, o_ref):
    o_ref[...] = _rms(moe_ref[...], pw_ref[...]) + h_ref[...]


def _final(moe, pw, h):
    blk = pl.BlockSpec((_BT, _D), lambda i: (i, 0))
    return pl.pallas_call(
        _final_body,
        grid=(_NT,),
        in_specs=[blk, pl.BlockSpec((1, _D), lambda i: (0, 0)), blk],
        out_specs=blk,
        out_shape=jax.ShapeDtypeStruct((_S, _D), jnp.float32),
    )(moe, pw, h)


def _final2_body(y0_ref, y1_ref, w0_ref, w1_ref, pw_ref, h_ref, o_ref):
    moe = w0_ref[...] * y0_ref[...] + w1_ref[...] * y1_ref[...]
    o_ref[...] = _rms(moe, pw_ref[...]) + h_ref[...]


def _final2(y0, y1, w0, w1, pw, h):
    blk = pl.BlockSpec((_BT, _D), lambda i: (i, 0))
    col = pl.BlockSpec((_BT, 1), lambda i: (i, 0))
    return pl.pallas_call(
        _final2_body,
        grid=(_NT,),
        in_specs=[blk, blk, col, col,
                  pl.BlockSpec((1, _D), lambda i: (0, 0)), blk],
        out_specs=blk,
        out_shape=jax.ShapeDtypeStruct((_S, _D), jnp.float32),
    )(y0, y1, w0, w1, pw, h)


def kernel(positions, hidden_states, residual, Wq, Wk, Wv, Wo, q_norm_w,
           k_norm_w, post_attn_w, post_ff_w, gate_w, w_gate, w_up, w_down):
    del residual
    x = hidden_states.reshape(_S, _D)
    cosf, sinf = _rope_tables(positions)
    qn = q_norm_w.reshape(1, _D)
    kn = k_norm_w.reshape(1, _D)
    paw = post_attn_w.reshape(1, _D)
    pfw = post_ff_w.reshape(1, _D)
    gw_pad = jnp.pad(gate_w, ((0, 0), (0, 128 - _E)))

    q, k, v = _qkv(x, Wq, Wk, Wv, qn, kn, cosf, sinf)
    qh = q.reshape(_S, _H, _DH).transpose(1, 0, 2)
    kh = k.reshape(_S, _H, _DH).transpose(1, 0, 2)
    vh = v.reshape(_S, _H, _DH).transpose(1, 0, 2)
    ctx = _attention(qh, kh, vh).transpose(1, 0, 2).reshape(_S, _D)
    h = _oproj(ctx, Wo, paw, x)
    wpad128, i1c, i2c, m1c, m2c = _router(h, gw_pad)
    moe = _moe(h, wpad128, w_gate, w_up, w_down)
    out = _final(moe, pfw, h)
    return out.reshape(_B, _S, _D)
